# baseline (speedup 1.0000x reference)
import functools
import jax
import jax.numpy as jnp
from jax.experimental import pallas as pl

RADIUS2 = 0.12 * 0.12
K = 16
TN = 256


def _project_kv_body(kv_ref, w_ref, b_ref, out_ref):
    kv = kv_ref[0]
    out = jnp.dot(kv, w_ref[...], preferred_element_type=jnp.float32, precision=jax.lax.Precision.HIGHEST)
    out_ref[0] = (out + b_ref[...]).astype(jnp.bfloat16)


def _fused_body(qx_ref, kvxT_ref, qf_ref, sv_ref, wta_ref, bta_ref,
                wff1_ref, bff1_ref, wff2_ref, bff2_ref,
                ln1g_ref, ln1b_ref, ln2g_ref, ln2b_ref, out_ref):
    C = qf_ref.shape[-1]
    qx = qx_ref[0]
    kvxT = kvxT_ref[0]
    qf = qf_ref[0]
    sv = sv_ref[0]

    cross = (jnp.dot(qx[:, 0:8], kvxT[0:8, :], preferred_element_type=jnp.float32)
             + jnp.dot(qx[:, 8:16], kvxT[8:16, :], preferred_element_type=jnp.float32))
    qfull = qx[:, 0:3] + qx[:, 8:11]
    qsq = jnp.sum(qfull * qfull, axis=1, keepdims=True)
    kfull = kvxT[0:3, :] + kvxT[11:14, :]
    ksq = jnp.sum(kfull * kfull, axis=0, keepdims=True)
    d2 = qsq + ksq - 2.0 * cross
    d2 = jnp.where(d2 <= RADIUS2, d2, jnp.float32(1e30))

    t = jnp.dot(qf, wta_ref[...], preferred_element_type=jnp.float32, precision=jax.lax.Precision.HIGHEST)
    t = t + bta_ref[...]

    alphas = []
    vals = []
    mprev = jnp.full((qx.shape[0], 1), -1.0, jnp.float32)
    for k in range(K):
        if k == 0:
            dm = d2
        else:
            dm = jnp.where(d2 > mprev, d2, jnp.float32(1e30))
        m = jnp.min(dm, axis=1, keepdims=True)
        oh = dm == m
        g = jnp.dot(oh.astype(jnp.bfloat16), sv,
                    preferred_element_type=jnp.float32)
        s_k = g[:, :C]
        v_k = g[:, C:]
        valid = m <= RADIUS2
        alpha = jnp.where(valid, jnp.maximum(t - s_k, 0.0), -1e30)
        alphas.append(alpha)
        vals.append(v_k)
        mprev = m

    mx = alphas[0]
    for a in alphas[1:]:
        mx = jnp.maximum(mx, a)
    mx = jnp.maximum(mx, 0.0)
    den = jnp.zeros_like(mx)
    acc = jnp.zeros_like(mx)
    for a, v in zip(alphas, vals):
        e = jnp.exp(a - mx)
        den = den + e
        acc = acc + e * v
    conv = acc / jnp.maximum(den, 1e-16)

    x1 = qf + conv
    mu = jnp.mean(x1, axis=1, keepdims=True)
    var = jnp.mean((x1 - mu) ** 2, axis=1, keepdims=True)
    x1 = (x1 - mu) * jax.lax.rsqrt(var + 1e-5) * ln1g_ref[...] + ln1b_ref[...]

    h = jnp.dot(x1.astype(jnp.bfloat16), wff1_ref[...].astype(jnp.bfloat16), preferred_element_type=jnp.float32)
    h = h + bff1_ref[...]
    h = 0.5 * h * (1.0 + jax.lax.erf(h * 0.7071067811865476))
    h2 = jnp.dot(h.astype(jnp.bfloat16), wff2_ref[...].astype(jnp.bfloat16), preferred_element_type=jnp.float32)
    h2 = h2 + bff2_ref[...]

    x2 = x1 + h2
    mu2 = jnp.mean(x2, axis=1, keepdims=True)
    var2 = jnp.mean((x2 - mu2) ** 2, axis=1, keepdims=True)
    out_ref[0] = (x2 - mu2) * jax.lax.rsqrt(var2 + 1e-5) * ln2g_ref[...] + ln2b_ref[...]


@jax.jit
def _run(q_xyz, q_feat, kv_xyz, kv_feat, W_lin, b_lin, W_src, b_src,
         W_dst, b_dst, W_attn, b_attn, ln1_g, ln1_b, W_ff1, b_ff1,
         W_ff2, b_ff2, ln2_g, ln2_b):
    B, N, C = q_feat.shape
    L = kv_xyz.shape[1]
    F = W_ff1.shape[1]

    W_sa = W_src @ W_attn
    b_sa = b_src @ W_attn
    W_ta = W_dst @ W_attn
    b_ta = b_dst @ W_attn + b_attn
    Wcat = jnp.concatenate([W_sa, W_lin], axis=1)
    bcat = jnp.concatenate([b_sa, b_lin])[None, :]

    qh = q_xyz.astype(jnp.bfloat16).astype(jnp.float32)
    ql = (q_xyz - qh).astype(jnp.bfloat16).astype(jnp.float32)
    z5q = jnp.zeros(qh.shape[:-1] + (5,), jnp.float32)
    qx = jnp.pad(jnp.concatenate([qh, z5q, ql, qh], axis=-1),
                 ((0, 0), (0, 0), (0, 128 - 14)))
    kh = kv_xyz.astype(jnp.bfloat16).astype(jnp.float32)
    kl = (kv_xyz - kh).astype(jnp.bfloat16).astype(jnp.float32)
    z5k = jnp.zeros(kh.shape[:-1] + (5,), jnp.float32)
    kvxT = jnp.pad(
        jnp.swapaxes(jnp.concatenate([kh, z5k, kh, kl], axis=-1), 1, 2),
        ((0, 0), (0, 128 - 14), (0, 0)))

    sv = pl.pallas_call(
        _project_kv_body,
        grid=(B,),
        in_specs=[
            pl.BlockSpec((1, L, C), lambda b: (b, 0, 0)),
            pl.BlockSpec((C, 2 * C), lambda b: (0, 0)),
            pl.BlockSpec((1, 2 * C), lambda b: (0, 0)),
        ],
        out_specs=pl.BlockSpec((1, L, 2 * C), lambda b: (b, 0, 0)),
        out_shape=jax.ShapeDtypeStruct((B, L, 2 * C), jnp.bfloat16),
    )(kv_feat, Wcat, bcat)

    grid = (B, N // TN)
    out = pl.pallas_call(
        _fused_body,
        grid=grid,
        in_specs=[
            pl.BlockSpec((1, TN, 128), lambda b, n: (b, n, 0)),
            pl.BlockSpec((1, 128, L), lambda b, n: (b, 0, 0)),
            pl.BlockSpec((1, TN, C), lambda b, n: (b, n, 0)),
            pl.BlockSpec((1, L, 2 * C), lambda b, n: (b, 0, 0)),
            pl.BlockSpec((C, C), lambda b, n: (0, 0)),
            pl.BlockSpec((1, C), lambda b, n: (0, 0)),
            pl.BlockSpec((C, F), lambda b, n: (0, 0)),
            pl.BlockSpec((1, F), lambda b, n: (0, 0)),
            pl.BlockSpec((F, C), lambda b, n: (0, 0)),
            pl.BlockSpec((1, C), lambda b, n: (0, 0)),
            pl.BlockSpec((1, C), lambda b, n: (0, 0)),
            pl.BlockSpec((1, C), lambda b, n: (0, 0)),
            pl.BlockSpec((1, C), lambda b, n: (0, 0)),
            pl.BlockSpec((1, C), lambda b, n: (0, 0)),
        ],
        out_specs=pl.BlockSpec((1, TN, C), lambda b, n: (b, n, 0)),
        out_shape=jax.ShapeDtypeStruct((B, N, C), jnp.float32),
    )(qx, kvxT, q_feat, sv, W_ta, b_ta[None, :], W_ff1, b_ff1[None, :],
      W_ff2, b_ff2[None, :], ln1_g[None, :], ln1_b[None, :],
      ln2_g[None, :], ln2_b[None, :])
    return out


def kernel(q_xyz, q_feat, kv_xyz, kv_feat, W_lin, b_lin, W_src, b_src,
           W_dst, b_dst, W_attn, b_attn, ln1_g, ln1_b, W_ff1, b_ff1,
           W_ff2, b_ff2, ln2_g, ln2_b):
    return _run(q_xyz, q_feat, kv_xyz, kv_feat, W_lin, b_lin, W_src, b_src,
                W_dst, b_dst, W_attn, b_attn, ln1_g, ln1_b, W_ff1, b_ff1,
                W_ff2, b_ff2, ln2_g, ln2_b)


# speedup vs baseline: 1.0198x; 1.0198x over previous
import functools
import jax
import jax.numpy as jnp
from jax.experimental import pallas as pl

RADIUS2 = 0.12 * 0.12
K = 16
TN = 256


def _project_kv_body(kv_ref, w_ref, b_ref, out_ref):
    kv = kv_ref[0]
    out = jnp.dot(kv, w_ref[...], preferred_element_type=jnp.float32, precision=jax.lax.Precision.HIGHEST)
    out_ref[0] = (out + b_ref[...]).astype(jnp.bfloat16)


def _fused_body(qx_ref, kvxT_ref, qf_ref, sv_ref, wta_ref, bta_ref,
                wff1_ref, bff1_ref, wff2_ref, bff2_ref,
                ln1g_ref, ln1b_ref, ln2g_ref, ln2b_ref, out_ref):
    C = qf_ref.shape[-1]
    qx = qx_ref[0]
    kvxT = kvxT_ref[0]
    qf = qf_ref[0]
    sv = sv_ref[0]

    cross = (jnp.dot(qx[:, 0:8], kvxT[0:8, :], preferred_element_type=jnp.float32)
             + jnp.dot(qx[:, 8:16], kvxT[8:16, :], preferred_element_type=jnp.float32))
    qfull = qx[:, 0:3] + qx[:, 8:11]
    qsq = jnp.sum(qfull * qfull, axis=1, keepdims=True)
    kfull = kvxT[0:3, :] + kvxT[11:14, :]
    ksq = jnp.sum(kfull * kfull, axis=0, keepdims=True)
    d2 = qsq + ksq - 2.0 * cross
    d2 = jnp.where(d2 <= RADIUS2, d2, jnp.float32(1e30))

    t = jnp.dot(qf, wta_ref[...], preferred_element_type=jnp.float32, precision=jax.lax.Precision.HIGHEST)
    t = t + bta_ref[...]

    shape = (qx.shape[0], C)
    mx = jnp.zeros(shape, jnp.float32)
    den = jnp.zeros(shape, jnp.float32)
    acc = jnp.zeros(shape, jnp.float32)
    mprev = jnp.full((qx.shape[0], 1), -1.0, jnp.float32)
    for k in range(K):
        if k == 0:
            m = jnp.min(d2, axis=1, keepdims=True)
        else:
            m = jnp.min(jnp.where(d2 > mprev, d2, jnp.float32(1e30)),
                        axis=1, keepdims=True)
        oh = d2 == m
        g = jnp.dot(oh.astype(jnp.bfloat16), sv,
                    preferred_element_type=jnp.float32)
        s_k = g[:, :C]
        v_k = g[:, C:]
        valid = m <= RADIUS2
        alpha = jnp.maximum(t - s_k, 0.0)
        mx_new = jnp.where(valid, jnp.maximum(mx, alpha), mx)
        scale = jnp.exp(mx - mx_new)
        e = jnp.where(valid, jnp.exp(alpha - mx_new), 0.0)
        den = den * scale + e
        acc = acc * scale + e * v_k
        mx = mx_new
        mprev = m
    conv = acc / jnp.maximum(den, 1e-16)

    x1 = qf + conv
    mu = jnp.mean(x1, axis=1, keepdims=True)
    var = jnp.mean((x1 - mu) ** 2, axis=1, keepdims=True)
    x1 = (x1 - mu) * jax.lax.rsqrt(var + 1e-5) * ln1g_ref[...] + ln1b_ref[...]

    h = jnp.dot(x1.astype(jnp.bfloat16), wff1_ref[...].astype(jnp.bfloat16), preferred_element_type=jnp.float32)
    h = h + bff1_ref[...]
    h = 0.5 * h * (1.0 + jax.lax.erf(h * 0.7071067811865476))
    h2 = jnp.dot(h.astype(jnp.bfloat16), wff2_ref[...].astype(jnp.bfloat16), preferred_element_type=jnp.float32)
    h2 = h2 + bff2_ref[...]

    x2 = x1 + h2
    mu2 = jnp.mean(x2, axis=1, keepdims=True)
    var2 = jnp.mean((x2 - mu2) ** 2, axis=1, keepdims=True)
    out_ref[0] = (x2 - mu2) * jax.lax.rsqrt(var2 + 1e-5) * ln2g_ref[...] + ln2b_ref[...]


@jax.jit
def _run(q_xyz, q_feat, kv_xyz, kv_feat, W_lin, b_lin, W_src, b_src,
         W_dst, b_dst, W_attn, b_attn, ln1_g, ln1_b, W_ff1, b_ff1,
         W_ff2, b_ff2, ln2_g, ln2_b):
    B, N, C = q_feat.shape
    L = kv_xyz.shape[1]
    F = W_ff1.shape[1]

    W_sa = W_src @ W_attn
    b_sa = b_src @ W_attn
    W_ta = W_dst @ W_attn
    b_ta = b_dst @ W_attn + b_attn
    Wcat = jnp.concatenate([W_sa, W_lin], axis=1)
    bcat = jnp.concatenate([b_sa, b_lin])[None, :]

    qh = q_xyz.astype(jnp.bfloat16).astype(jnp.float32)
    ql = (q_xyz - qh).astype(jnp.bfloat16).astype(jnp.float32)
    z5q = jnp.zeros(qh.shape[:-1] + (5,), jnp.float32)
    qx = jnp.pad(jnp.concatenate([qh, z5q, ql, qh], axis=-1),
                 ((0, 0), (0, 0), (0, 128 - 14)))
    kh = kv_xyz.astype(jnp.bfloat16).astype(jnp.float32)
    kl = (kv_xyz - kh).astype(jnp.bfloat16).astype(jnp.float32)
    z5k = jnp.zeros(kh.shape[:-1] + (5,), jnp.float32)
    kvxT = jnp.pad(
        jnp.swapaxes(jnp.concatenate([kh, z5k, kh, kl], axis=-1), 1, 2),
        ((0, 0), (0, 128 - 14), (0, 0)))

    sv = pl.pallas_call(
        _project_kv_body,
        grid=(B,),
        in_specs=[
            pl.BlockSpec((1, L, C), lambda b: (b, 0, 0)),
            pl.BlockSpec((C, 2 * C), lambda b: (0, 0)),
            pl.BlockSpec((1, 2 * C), lambda b: (0, 0)),
        ],
        out_specs=pl.BlockSpec((1, L, 2 * C), lambda b: (b, 0, 0)),
        out_shape=jax.ShapeDtypeStruct((B, L, 2 * C), jnp.bfloat16),
    )(kv_feat, Wcat, bcat)

    grid = (B, N // TN)
    out = pl.pallas_call(
        _fused_body,
        grid=grid,
        in_specs=[
            pl.BlockSpec((1, TN, 128), lambda b, n: (b, n, 0)),
            pl.BlockSpec((1, 128, L), lambda b, n: (b, 0, 0)),
            pl.BlockSpec((1, TN, C), lambda b, n: (b, n, 0)),
            pl.BlockSpec((1, L, 2 * C), lambda b, n: (b, 0, 0)),
            pl.BlockSpec((C, C), lambda b, n: (0, 0)),
            pl.BlockSpec((1, C), lambda b, n: (0, 0)),
            pl.BlockSpec((C, F), lambda b, n: (0, 0)),
            pl.BlockSpec((1, F), lambda b, n: (0, 0)),
            pl.BlockSpec((F, C), lambda b, n: (0, 0)),
            pl.BlockSpec((1, C), lambda b, n: (0, 0)),
            pl.BlockSpec((1, C), lambda b, n: (0, 0)),
            pl.BlockSpec((1, C), lambda b, n: (0, 0)),
            pl.BlockSpec((1, C), lambda b, n: (0, 0)),
            pl.BlockSpec((1, C), lambda b, n: (0, 0)),
        ],
        out_specs=pl.BlockSpec((1, TN, C), lambda b, n: (b, n, 0)),
        out_shape=jax.ShapeDtypeStruct((B, N, C), jnp.float32),
    )(qx, kvxT, q_feat, sv, W_ta, b_ta[None, :], W_ff1, b_ff1[None, :],
      W_ff2, b_ff2[None, :], ln1_g[None, :], ln1_b[None, :],
      ln2_g[None, :], ln2_b[None, :])
    return out


def kernel(q_xyz, q_feat, kv_xyz, kv_feat, W_lin, b_lin, W_src, b_src,
           W_dst, b_dst, W_attn, b_attn, ln1_g, ln1_b, W_ff1, b_ff1,
           W_ff2, b_ff2, ln2_g, ln2_b):
    return _run(q_xyz, q_feat, kv_xyz, kv_feat, W_lin, b_lin, W_src, b_src,
                W_dst, b_dst, W_attn, b_attn, ln1_g, ln1_b, W_ff1, b_ff1,
                W_ff2, b_ff2, ln2_g, ln2_b)


# speedup vs baseline: 1.0505x; 1.0301x over previous
import functools
import jax
import jax.numpy as jnp
from jax.experimental import pallas as pl

RADIUS2 = 0.12 * 0.12
K = 16
TN = 256


def _project_kv_body(kv_ref, w_ref, b_ref, out_ref):
    kv = kv_ref[0]
    out = jnp.dot(kv, w_ref[...], preferred_element_type=jnp.float32, precision=jax.lax.Precision.HIGHEST)
    out_ref[0] = out + b_ref[...]


def _fused_body(qx_ref, kvxT_ref, qf_ref, sv_ref, wta_ref, bta_ref,
                wff1_ref, bff1_ref, wff2_ref, bff2_ref,
                ln1g_ref, ln1b_ref, ln2g_ref, ln2b_ref, out_ref):
    C = qf_ref.shape[-1]
    qx = qx_ref[0]
    kvxT = kvxT_ref[0]
    qf = qf_ref[0]
    sv = sv_ref[0]

    cross = (jnp.dot(qx[:, 0:8], kvxT[0:8, :], preferred_element_type=jnp.float32)
             + jnp.dot(qx[:, 8:16], kvxT[8:16, :], preferred_element_type=jnp.float32))
    qfull = qx[:, 0:3] + qx[:, 8:11]
    qsq = jnp.sum(qfull * qfull, axis=1, keepdims=True)
    kfull = kvxT[0:3, :] + kvxT[11:14, :]
    ksq = jnp.sum(kfull * kfull, axis=0, keepdims=True)
    d2 = qsq + ksq - 2.0 * cross
    d2 = jnp.where(d2 <= RADIUS2, d2, jnp.float32(1e30))

    t = jnp.dot(qf, wta_ref[...], preferred_element_type=jnp.float32, precision=jax.lax.Precision.HIGHEST)
    t = t + bta_ref[...]

    shape = (qx.shape[0], C)
    mx = jnp.zeros(shape, jnp.float32)
    den = jnp.zeros(shape, jnp.float32)
    acc = jnp.zeros(shape, jnp.float32)
    mprev = jnp.full((qx.shape[0], 1), -1.0, jnp.float32)
    for k in range(K):
        if k == 0:
            m = jnp.min(d2, axis=1, keepdims=True)
        else:
            m = jnp.min(jnp.where(d2 > mprev, d2, jnp.float32(1e30)),
                        axis=1, keepdims=True)
        oh = d2 == m
        g = jnp.dot(oh.astype(jnp.float32), sv,
                    preferred_element_type=jnp.float32)
        s_k = g[:, :C]
        v_k = g[:, C:]
        valid = m <= RADIUS2
        alpha = jnp.maximum(t - s_k, 0.0)
        mx_new = jnp.where(valid, jnp.maximum(mx, alpha), mx)
        scale = jnp.exp(mx - mx_new)
        e = jnp.where(valid, jnp.exp(alpha - mx_new), 0.0)
        den = den * scale + e
        acc = acc * scale + e * v_k
        mx = mx_new
        mprev = m
    conv = acc / jnp.maximum(den, 1e-16)

    x1 = qf + conv
    mu = jnp.mean(x1, axis=1, keepdims=True)
    var = jnp.mean((x1 - mu) ** 2, axis=1, keepdims=True)
    x1 = (x1 - mu) * jax.lax.rsqrt(var + 1e-5) * ln1g_ref[...] + ln1b_ref[...]

    h = jnp.dot(x1.astype(jnp.bfloat16), wff1_ref[...].astype(jnp.bfloat16), preferred_element_type=jnp.float32)
    h = h + bff1_ref[...]
    h = 0.5 * h * (1.0 + jax.lax.erf(h * 0.7071067811865476))
    h2 = jnp.dot(h.astype(jnp.bfloat16), wff2_ref[...].astype(jnp.bfloat16), preferred_element_type=jnp.float32)
    h2 = h2 + bff2_ref[...]

    x2 = x1 + h2
    mu2 = jnp.mean(x2, axis=1, keepdims=True)
    var2 = jnp.mean((x2 - mu2) ** 2, axis=1, keepdims=True)
    out_ref[0] = (x2 - mu2) * jax.lax.rsqrt(var2 + 1e-5) * ln2g_ref[...] + ln2b_ref[...]


@jax.jit
def _run(q_xyz, q_feat, kv_xyz, kv_feat, W_lin, b_lin, W_src, b_src,
         W_dst, b_dst, W_attn, b_attn, ln1_g, ln1_b, W_ff1, b_ff1,
         W_ff2, b_ff2, ln2_g, ln2_b):
    B, N, C = q_feat.shape
    L = kv_xyz.shape[1]
    F = W_ff1.shape[1]

    W_sa = W_src @ W_attn
    b_sa = b_src @ W_attn
    W_ta = W_dst @ W_attn
    b_ta = b_dst @ W_attn + b_attn
    Wcat = jnp.concatenate([W_sa, W_lin], axis=1)
    bcat = jnp.concatenate([b_sa, b_lin])[None, :]

    qh = q_xyz.astype(jnp.bfloat16).astype(jnp.float32)
    ql = (q_xyz - qh).astype(jnp.bfloat16).astype(jnp.float32)
    z5q = jnp.zeros(qh.shape[:-1] + (5,), jnp.float32)
    qx = jnp.pad(jnp.concatenate([qh, z5q, ql, qh], axis=-1),
                 ((0, 0), (0, 0), (0, 128 - 14)))
    kh = kv_xyz.astype(jnp.bfloat16).astype(jnp.float32)
    kl = (kv_xyz - kh).astype(jnp.bfloat16).astype(jnp.float32)
    z5k = jnp.zeros(kh.shape[:-1] + (5,), jnp.float32)
    kvxT = jnp.pad(
        jnp.swapaxes(jnp.concatenate([kh, z5k, kh, kl], axis=-1), 1, 2),
        ((0, 0), (0, 128 - 14), (0, 0)))

    sv = pl.pallas_call(
        _project_kv_body,
        grid=(B,),
        in_specs=[
            pl.BlockSpec((1, L, C), lambda b: (b, 0, 0)),
            pl.BlockSpec((C, 2 * C), lambda b: (0, 0)),
            pl.BlockSpec((1, 2 * C), lambda b: (0, 0)),
        ],
        out_specs=pl.BlockSpec((1, L, 2 * C), lambda b: (b, 0, 0)),
        out_shape=jax.ShapeDtypeStruct((B, L, 2 * C), jnp.float32),
    )(kv_feat, Wcat, bcat)

    grid = (B, N // TN)
    out = pl.pallas_call(
        _fused_body,
        grid=grid,
        in_specs=[
            pl.BlockSpec((1, TN, 128), lambda b, n: (b, n, 0)),
            pl.BlockSpec((1, 128, L), lambda b, n: (b, 0, 0)),
            pl.BlockSpec((1, TN, C), lambda b, n: (b, n, 0)),
            pl.BlockSpec((1, L, 2 * C), lambda b, n: (b, 0, 0)),
            pl.BlockSpec((C, C), lambda b, n: (0, 0)),
            pl.BlockSpec((1, C), lambda b, n: (0, 0)),
            pl.BlockSpec((C, F), lambda b, n: (0, 0)),
            pl.BlockSpec((1, F), lambda b, n: (0, 0)),
            pl.BlockSpec((F, C), lambda b, n: (0, 0)),
            pl.BlockSpec((1, C), lambda b, n: (0, 0)),
            pl.BlockSpec((1, C), lambda b, n: (0, 0)),
            pl.BlockSpec((1, C), lambda b, n: (0, 0)),
            pl.BlockSpec((1, C), lambda b, n: (0, 0)),
            pl.BlockSpec((1, C), lambda b, n: (0, 0)),
        ],
        out_specs=pl.BlockSpec((1, TN, C), lambda b, n: (b, n, 0)),
        out_shape=jax.ShapeDtypeStruct((B, N, C), jnp.float32),
    )(qx, kvxT, q_feat, sv, W_ta, b_ta[None, :], W_ff1, b_ff1[None, :],
      W_ff2, b_ff2[None, :], ln1_g[None, :], ln1_b[None, :],
      ln2_g[None, :], ln2_b[None, :])
    return out


def kernel(q_xyz, q_feat, kv_xyz, kv_feat, W_lin, b_lin, W_src, b_src,
           W_dst, b_dst, W_attn, b_attn, ln1_g, ln1_b, W_ff1, b_ff1,
           W_ff2, b_ff2, ln2_g, ln2_b):
    return _run(q_xyz, q_feat, kv_xyz, kv_feat, W_lin, b_lin, W_src, b_src,
                W_dst, b_dst, W_attn, b_attn, ln1_g, ln1_b, W_ff1, b_ff1,
                W_ff2, b_ff2, ln2_g, ln2_b)


# speedup vs baseline: 1.0619x; 1.0108x over previous
import functools
import jax
import jax.numpy as jnp
from jax.experimental import pallas as pl

RADIUS2 = 0.12 * 0.12
K = 16
TN = 512


def _project_kv_body(kv_ref, w_ref, b_ref, out_ref):
    kv = kv_ref[0]
    out = jnp.dot(kv, w_ref[...], preferred_element_type=jnp.float32, precision=jax.lax.Precision.HIGHEST)
    out_ref[0] = out + b_ref[...]


def _fused_body(qx_ref, kvxT_ref, qf_ref, sv_ref, wta_ref, bta_ref,
                wff1_ref, bff1_ref, wff2_ref, bff2_ref,
                ln1g_ref, ln1b_ref, ln2g_ref, ln2b_ref, out_ref):
    C = qf_ref.shape[-1]
    qx = qx_ref[0]
    kvxT = kvxT_ref[0]
    qf = qf_ref[0]
    sv = sv_ref[0]

    cross = (jnp.dot(qx[:, 0:8], kvxT[0:8, :], preferred_element_type=jnp.float32)
             + jnp.dot(qx[:, 8:16], kvxT[8:16, :], preferred_element_type=jnp.float32))
    qfull = qx[:, 0:3] + qx[:, 8:11]
    qsq = jnp.sum(qfull * qfull, axis=1, keepdims=True)
    kfull = kvxT[0:3, :] + kvxT[11:14, :]
    ksq = jnp.sum(kfull * kfull, axis=0, keepdims=True)
    d2 = qsq + ksq - 2.0 * cross
    d2 = jnp.where(d2 <= RADIUS2, d2, jnp.float32(1e30))

    t = jnp.dot(qf, wta_ref[...], preferred_element_type=jnp.float32, precision=jax.lax.Precision.HIGHEST)
    t = t + bta_ref[...]

    shape = (qx.shape[0], C)
    mx = jnp.zeros(shape, jnp.float32)
    den = jnp.zeros(shape, jnp.float32)
    acc = jnp.zeros(shape, jnp.float32)
    mprev = jnp.full((qx.shape[0], 1), -1.0, jnp.float32)
    for k in range(K):
        if k == 0:
            m = jnp.min(d2, axis=1, keepdims=True)
        else:
            m = jnp.min(jnp.where(d2 > mprev, d2, jnp.float32(1e30)),
                        axis=1, keepdims=True)
        oh = d2 == m
        g = jnp.dot(oh.astype(jnp.float32), sv,
                    preferred_element_type=jnp.float32)
        s_k = g[:, :C]
        v_k = g[:, C:]
        valid = m <= RADIUS2
        alpha = jnp.maximum(t - s_k, 0.0)
        mx_new = jnp.where(valid, jnp.maximum(mx, alpha), mx)
        scale = jnp.exp(mx - mx_new)
        e = jnp.where(valid, jnp.exp(alpha - mx_new), 0.0)
        den = den * scale + e
        acc = acc * scale + e * v_k
        mx = mx_new
        mprev = m
    conv = acc / jnp.maximum(den, 1e-16)

    x1 = qf + conv
    mu = jnp.mean(x1, axis=1, keepdims=True)
    var = jnp.mean((x1 - mu) ** 2, axis=1, keepdims=True)
    x1 = (x1 - mu) * jax.lax.rsqrt(var + 1e-5) * ln1g_ref[...] + ln1b_ref[...]

    h = jnp.dot(x1.astype(jnp.bfloat16), wff1_ref[...].astype(jnp.bfloat16), preferred_element_type=jnp.float32)
    h = h + bff1_ref[...]
    h = 0.5 * h * (1.0 + jax.lax.erf(h * 0.7071067811865476))
    h2 = jnp.dot(h.astype(jnp.bfloat16), wff2_ref[...].astype(jnp.bfloat16), preferred_element_type=jnp.float32)
    h2 = h2 + bff2_ref[...]

    x2 = x1 + h2
    mu2 = jnp.mean(x2, axis=1, keepdims=True)
    var2 = jnp.mean((x2 - mu2) ** 2, axis=1, keepdims=True)
    out_ref[0] = (x2 - mu2) * jax.lax.rsqrt(var2 + 1e-5) * ln2g_ref[...] + ln2b_ref[...]


@jax.jit
def _run(q_xyz, q_feat, kv_xyz, kv_feat, W_lin, b_lin, W_src, b_src,
         W_dst, b_dst, W_attn, b_attn, ln1_g, ln1_b, W_ff1, b_ff1,
         W_ff2, b_ff2, ln2_g, ln2_b):
    B, N, C = q_feat.shape
    L = kv_xyz.shape[1]
    F = W_ff1.shape[1]

    W_sa = W_src @ W_attn
    b_sa = b_src @ W_attn
    W_ta = W_dst @ W_attn
    b_ta = b_dst @ W_attn + b_attn
    Wcat = jnp.concatenate([W_sa, W_lin], axis=1)
    bcat = jnp.concatenate([b_sa, b_lin])[None, :]

    qh = q_xyz.astype(jnp.bfloat16).astype(jnp.float32)
    ql = (q_xyz - qh).astype(jnp.bfloat16).astype(jnp.float32)
    z5q = jnp.zeros(qh.shape[:-1] + (5,), jnp.float32)
    qx = jnp.pad(jnp.concatenate([qh, z5q, ql, qh], axis=-1),
                 ((0, 0), (0, 0), (0, 128 - 14)))
    kh = kv_xyz.astype(jnp.bfloat16).astype(jnp.float32)
    kl = (kv_xyz - kh).astype(jnp.bfloat16).astype(jnp.float32)
    z5k = jnp.zeros(kh.shape[:-1] + (5,), jnp.float32)
    kvxT = jnp.pad(
        jnp.swapaxes(jnp.concatenate([kh, z5k, kh, kl], axis=-1), 1, 2),
        ((0, 0), (0, 128 - 14), (0, 0)))

    sv = pl.pallas_call(
        _project_kv_body,
        grid=(B,),
        in_specs=[
            pl.BlockSpec((1, L, C), lambda b: (b, 0, 0)),
            pl.BlockSpec((C, 2 * C), lambda b: (0, 0)),
            pl.BlockSpec((1, 2 * C), lambda b: (0, 0)),
        ],
        out_specs=pl.BlockSpec((1, L, 2 * C), lambda b: (b, 0, 0)),
        out_shape=jax.ShapeDtypeStruct((B, L, 2 * C), jnp.float32),
    )(kv_feat, Wcat, bcat)

    grid = (B, N // TN)
    out = pl.pallas_call(
        _fused_body,
        grid=grid,
        in_specs=[
            pl.BlockSpec((1, TN, 128), lambda b, n: (b, n, 0)),
            pl.BlockSpec((1, 128, L), lambda b, n: (b, 0, 0)),
            pl.BlockSpec((1, TN, C), lambda b, n: (b, n, 0)),
            pl.BlockSpec((1, L, 2 * C), lambda b, n: (b, 0, 0)),
            pl.BlockSpec((C, C), lambda b, n: (0, 0)),
            pl.BlockSpec((1, C), lambda b, n: (0, 0)),
            pl.BlockSpec((C, F), lambda b, n: (0, 0)),
            pl.BlockSpec((1, F), lambda b, n: (0, 0)),
            pl.BlockSpec((F, C), lambda b, n: (0, 0)),
            pl.BlockSpec((1, C), lambda b, n: (0, 0)),
            pl.BlockSpec((1, C), lambda b, n: (0, 0)),
            pl.BlockSpec((1, C), lambda b, n: (0, 0)),
            pl.BlockSpec((1, C), lambda b, n: (0, 0)),
            pl.BlockSpec((1, C), lambda b, n: (0, 0)),
        ],
        out_specs=pl.BlockSpec((1, TN, C), lambda b, n: (b, n, 0)),
        out_shape=jax.ShapeDtypeStruct((B, N, C), jnp.float32),
    )(qx, kvxT, q_feat, sv, W_ta, b_ta[None, :], W_ff1, b_ff1[None, :],
      W_ff2, b_ff2[None, :], ln1_g[None, :], ln1_b[None, :],
      ln2_g[None, :], ln2_b[None, :])
    return out


def kernel(q_xyz, q_feat, kv_xyz, kv_feat, W_lin, b_lin, W_src, b_src,
           W_dst, b_dst, W_attn, b_attn, ln1_g, ln1_b, W_ff1, b_ff1,
           W_ff2, b_ff2, ln2_g, ln2_b):
    return _run(q_xyz, q_feat, kv_xyz, kv_feat, W_lin, b_lin, W_src, b_src,
                W_dst, b_dst, W_attn, b_attn, ln1_g, ln1_b, W_ff1, b_ff1,
                W_ff2, b_ff2, ln2_g, ln2_b)


# speedup vs baseline: 1.0907x; 1.0272x over previous
import functools
import jax
import jax.numpy as jnp
from jax.experimental import pallas as pl

RADIUS2 = 0.12 * 0.12
K = 16
TN = 512


def _project_kv_body(kv_ref, w_ref, b_ref, out_ref):
    kv = kv_ref[0]
    out = jnp.dot(kv, w_ref[...], preferred_element_type=jnp.float32, precision=jax.lax.Precision.HIGHEST)
    out_ref[0] = out + b_ref[...]


def _fused_body(qx_ref, kvxT_ref, qf_ref, sv_ref, wta_ref, bta_ref,
                wff1_ref, bff1_ref, wff2_ref, bff2_ref,
                ln1g_ref, ln1b_ref, ln2g_ref, ln2b_ref, out_ref):
    C = qf_ref.shape[-1]
    qx = qx_ref[0]
    kvxT = kvxT_ref[0]
    qf = qf_ref[0]
    sv = sv_ref[0]

    cross = (jnp.dot(qx[:, 0:8], kvxT[0:8, :], preferred_element_type=jnp.float32)
             + jnp.dot(qx[:, 8:16], kvxT[8:16, :], preferred_element_type=jnp.float32))
    qfull = qx[:, 0:3] + qx[:, 8:11]
    qsq = jnp.sum(qfull * qfull, axis=1, keepdims=True)
    kfull = kvxT[0:3, :] + kvxT[11:14, :]
    ksq = jnp.sum(kfull * kfull, axis=0, keepdims=True)
    d2 = qsq + ksq - 2.0 * cross
    d2 = jnp.where(d2 <= RADIUS2, d2, jnp.float32(1e30))

    t = jnp.dot(qf, wta_ref[...], preferred_element_type=jnp.float32, precision=jax.lax.Precision.HIGHEST)
    t = t + bta_ref[...]

    shape = (qx.shape[0], C)
    den = jnp.zeros(shape, jnp.float32)
    acc = jnp.zeros(shape, jnp.float32)
    mprev = jnp.full((qx.shape[0], 1), -1.0, jnp.float32)
    for k in range(K):
        if k == 0:
            m = jnp.min(d2, axis=1, keepdims=True)
        else:
            m = jnp.min(jnp.where(d2 > mprev, d2, jnp.float32(1e30)),
                        axis=1, keepdims=True)
        oh = d2 == m
        g = jnp.dot(oh.astype(jnp.float32), sv,
                    preferred_element_type=jnp.float32)
        s_k = g[:, :C]
        v_k = g[:, C:]
        valid = m <= RADIUS2
        alpha = jnp.maximum(t - s_k, 0.0)
        e = jnp.where(valid, jnp.exp(alpha), 0.0)
        den = den + e
        acc = acc + e * v_k
        mprev = m
    conv = acc / jnp.maximum(den, 1e-16)

    x1 = qf + conv
    mu = jnp.mean(x1, axis=1, keepdims=True)
    var = jnp.mean((x1 - mu) ** 2, axis=1, keepdims=True)
    x1 = (x1 - mu) * jax.lax.rsqrt(var + 1e-5) * ln1g_ref[...] + ln1b_ref[...]

    h = jnp.dot(x1.astype(jnp.bfloat16), wff1_ref[...].astype(jnp.bfloat16), preferred_element_type=jnp.float32)
    h = h + bff1_ref[...]
    h = 0.5 * h * (1.0 + jax.lax.erf(h * 0.7071067811865476))
    h2 = jnp.dot(h.astype(jnp.bfloat16), wff2_ref[...].astype(jnp.bfloat16), preferred_element_type=jnp.float32)
    h2 = h2 + bff2_ref[...]

    x2 = x1 + h2
    mu2 = jnp.mean(x2, axis=1, keepdims=True)
    var2 = jnp.mean((x2 - mu2) ** 2, axis=1, keepdims=True)
    out_ref[0] = (x2 - mu2) * jax.lax.rsqrt(var2 + 1e-5) * ln2g_ref[...] + ln2b_ref[...]


@jax.jit
def _run(q_xyz, q_feat, kv_xyz, kv_feat, W_lin, b_lin, W_src, b_src,
         W_dst, b_dst, W_attn, b_attn, ln1_g, ln1_b, W_ff1, b_ff1,
         W_ff2, b_ff2, ln2_g, ln2_b):
    B, N, C = q_feat.shape
    L = kv_xyz.shape[1]
    F = W_ff1.shape[1]

    W_sa = W_src @ W_attn
    b_sa = b_src @ W_attn
    W_ta = W_dst @ W_attn
    b_ta = b_dst @ W_attn + b_attn
    Wcat = jnp.concatenate([W_sa, W_lin], axis=1)
    bcat = jnp.concatenate([b_sa, b_lin])[None, :]

    qh = q_xyz.astype(jnp.bfloat16).astype(jnp.float32)
    ql = (q_xyz - qh).astype(jnp.bfloat16).astype(jnp.float32)
    z5q = jnp.zeros(qh.shape[:-1] + (5,), jnp.float32)
    qx = jnp.pad(jnp.concatenate([qh, z5q, ql, qh], axis=-1),
                 ((0, 0), (0, 0), (0, 128 - 14)))
    kh = kv_xyz.astype(jnp.bfloat16).astype(jnp.float32)
    kl = (kv_xyz - kh).astype(jnp.bfloat16).astype(jnp.float32)
    z5k = jnp.zeros(kh.shape[:-1] + (5,), jnp.float32)
    kvxT = jnp.pad(
        jnp.swapaxes(jnp.concatenate([kh, z5k, kh, kl], axis=-1), 1, 2),
        ((0, 0), (0, 128 - 14), (0, 0)))

    sv = pl.pallas_call(
        _project_kv_body,
        grid=(B,),
        in_specs=[
            pl.BlockSpec((1, L, C), lambda b: (b, 0, 0)),
            pl.BlockSpec((C, 2 * C), lambda b: (0, 0)),
            pl.BlockSpec((1, 2 * C), lambda b: (0, 0)),
        ],
        out_specs=pl.BlockSpec((1, L, 2 * C), lambda b: (b, 0, 0)),
        out_shape=jax.ShapeDtypeStruct((B, L, 2 * C), jnp.float32),
    )(kv_feat, Wcat, bcat)

    grid = (B, N // TN)
    out = pl.pallas_call(
        _fused_body,
        grid=grid,
        in_specs=[
            pl.BlockSpec((1, TN, 128), lambda b, n: (b, n, 0)),
            pl.BlockSpec((1, 128, L), lambda b, n: (b, 0, 0)),
            pl.BlockSpec((1, TN, C), lambda b, n: (b, n, 0)),
            pl.BlockSpec((1, L, 2 * C), lambda b, n: (b, 0, 0)),
            pl.BlockSpec((C, C), lambda b, n: (0, 0)),
            pl.BlockSpec((1, C), lambda b, n: (0, 0)),
            pl.BlockSpec((C, F), lambda b, n: (0, 0)),
            pl.BlockSpec((1, F), lambda b, n: (0, 0)),
            pl.BlockSpec((F, C), lambda b, n: (0, 0)),
            pl.BlockSpec((1, C), lambda b, n: (0, 0)),
            pl.BlockSpec((1, C), lambda b, n: (0, 0)),
            pl.BlockSpec((1, C), lambda b, n: (0, 0)),
            pl.BlockSpec((1, C), lambda b, n: (0, 0)),
            pl.BlockSpec((1, C), lambda b, n: (0, 0)),
        ],
        out_specs=pl.BlockSpec((1, TN, C), lambda b, n: (b, n, 0)),
        out_shape=jax.ShapeDtypeStruct((B, N, C), jnp.float32),
    )(qx, kvxT, q_feat, sv, W_ta, b_ta[None, :], W_ff1, b_ff1[None, :],
      W_ff2, b_ff2[None, :], ln1_g[None, :], ln1_b[None, :],
      ln2_g[None, :], ln2_b[None, :])
    return out


def kernel(q_xyz, q_feat, kv_xyz, kv_feat, W_lin, b_lin, W_src, b_src,
           W_dst, b_dst, W_attn, b_attn, ln1_g, ln1_b, W_ff1, b_ff1,
           W_ff2, b_ff2, ln2_g, ln2_b):
    return _run(q_xyz, q_feat, kv_xyz, kv_feat, W_lin, b_lin, W_src, b_src,
                W_dst, b_dst, W_attn, b_attn, ln1_g, ln1_b, W_ff1, b_ff1,
                W_ff2, b_ff2, ln2_g, ln2_b)
